# baseline (speedup 1.0000x reference)
import dataclasses
import functools

import jax
import jax.numpy as jnp
from jax import lax
from jax.experimental import pallas as pl
from jax.experimental.pallas import tpu as pltpu
from jax.experimental.pallas import tpu_sc as plsc

_B, _W, _H, _L = 4, 32, 32, 32
_N = _B * _W * _H * _L
_D = 256
_NUM_EMB = (16, 8, 4, 6)
_OFFS = (0, 16, 24, 28)
_TBL_ROWS = 40
_SENT_ROW = 34

_COUNT = 13107
_VPAD = 13312
_SENT_SLOT = 13200

_ROWS, _COLS = 512, 256

_NC, _NS, _LANES = 2, 16, 16
_NW = _NC * _NS
_TPH = _NW // 4
_TROWS = _ROWS // _TPH


def _rank_body(m0_ref, m1_ref, m2_ref, m3_ref,
               v0_ref, v1_ref, v2_ref, v3_ref, rank_ref, vp_ref):
    iota_r = lax.broadcasted_iota(jnp.int32, (_COLS, _COLS), 0)
    iota_c = lax.broadcasted_iota(jnp.int32, (_COLS, _COLS), 1)
    upper = (iota_r <= iota_c).astype(jnp.bfloat16)
    iota_r2 = lax.broadcasted_iota(jnp.int32, (_ROWS, _ROWS), 0)
    iota_c2 = lax.broadcasted_iota(jnp.int32, (_ROWS, _ROWS), 1)
    strict_lower = (iota_c2 < iota_r2).astype(jnp.bfloat16)

    masks = [m0_ref[...], m1_ref[...], m2_ref[...], m3_ref[...]]
    m_all = jnp.concatenate([m.astype(jnp.bfloat16) for m in masks], axis=0)
    row_incl_all = lax.dot(m_all, upper,
                           preferred_element_type=jnp.float32)
    tot_cols = jnp.concatenate(
        [row_incl_all[i * _ROWS:(i + 1) * _ROWS, _COLS - 1:_COLS]
         for i in range(4)], axis=1)
    hi = jnp.floor(tot_cols * (1.0 / 256.0))
    lo = tot_cols - hi * 256.0
    hilo = jnp.concatenate([hi, lo], axis=1).astype(jnp.bfloat16)
    scan8 = lax.dot(strict_lower, hilo,
                    preferred_element_type=jnp.float32)
    col_excl_all = scan8[:, :4] * 256.0 + scan8[:, 4:]
    for i, (m, v_ref) in enumerate(zip(masks, (v0_ref, v1_ref, v2_ref,
                                               v3_ref))):
        row_incl = row_incl_all[i * _ROWS:(i + 1) * _ROWS]
        excl = row_incl - m.astype(jnp.float32) + col_excl_all[:, i:i + 1]
        rank = excl.astype(jnp.int32)
        rank_ref[i] = jnp.where(m, rank, _SENT_SLOT)
        vp_ref[i] = jnp.concatenate(
            [v_ref[...],
             jnp.full((_VPAD - _COUNT,), _SENT_ROW - _OFFS[i], jnp.int32)])


def _ranks(masks, values):
    return pl.pallas_call(
        _rank_body,
        out_shape=(jax.ShapeDtypeStruct((4, _ROWS, _COLS), jnp.int32),
                   jax.ShapeDtypeStruct((4, _VPAD), jnp.int32)),
    )(*masks, *values)


def _sc_compiler_params():
    cp = pltpu.CompilerParams()
    if "needs_layout_passes" in pltpu.CompilerParams.__dataclass_fields__:
        cp = dataclasses.replace(cp, needs_layout_passes=False)
    return cp


def _gather_body(vals_hbm, idx_hbm, out_hbm, vals_v, idx_v, out_v, sem, sem2):
    wid = lax.axis_index("s") * _NC + lax.axis_index("c")
    head = wid // _TPH
    row0 = (wid % _TPH) * _TROWS
    c1 = pltpu.async_copy(vals_hbm.at[head], vals_v, sem)
    c2 = pltpu.async_copy(idx_hbm.at[head, pl.ds(row0, _TROWS)], idx_v, sem2)
    c1.wait()
    c2.wait()

    @plsc.parallel_loop(0, _TROWS, unroll=2)
    def _(r):
        for c in range(_COLS // _LANES):
            sl = pl.ds(c * _LANES, _LANES)
            out_v[r, sl] = plsc.load_gather(vals_v, [idx_v[r, sl]])

    pltpu.async_copy(out_v, out_hbm.at[head, pl.ds(row0, _TROWS)], sem).wait()


def _sc_gather(vals_pad, rank):
    mesh = plsc.VectorSubcoreMesh(core_axis_name="c", subcore_axis_name="s")
    k = pl.kernel(
        _gather_body,
        out_type=jax.ShapeDtypeStruct((4, _ROWS, _COLS), jnp.int32),
        mesh=mesh,
        scratch_types=[
            pltpu.VMEM((_VPAD,), jnp.int32),
            pltpu.VMEM((_TROWS, _COLS), jnp.int32),
            pltpu.VMEM((_TROWS, _COLS), jnp.int32),
            pltpu.SemaphoreType.DMA,
            pltpu.SemaphoreType.DMA,
        ],
        compiler_params=_sc_compiler_params(),
    )
    return k(vals_pad, rank)


_CR = 32


def _expand_body(sel_ref, tbl_ref, out_ref):
    iota40 = lax.broadcasted_iota(jnp.int32, (_TBL_ROWS, _COLS), 0)
    tbl = tbl_ref[...]
    for r in range(_CR):
        st = None
        for i in range(4):
            sel = (sel_ref[i, r, :] + _OFFS[i]).reshape(1, _COLS)
            eq = jnp.broadcast_to(sel, (_TBL_ROWS, _COLS)) == iota40
            st = eq.astype(jnp.int32) if st is None else st + eq
        tile = lax.dot_general(st.astype(jnp.float32), tbl,
                               (((0,), (0,)), ((), ())),
                               preferred_element_type=jnp.float32)
        out_ref[pl.ds(r * _COLS, _COLS), :] = tile


def _expand(sel, table40):
    return pl.pallas_call(
        _expand_body,
        grid=(_ROWS // _CR,),
        in_specs=[
            pl.BlockSpec((4, _CR, _COLS), lambda j: (0, j, 0)),
            pl.BlockSpec((_TBL_ROWS, _D), lambda j: (0, 0)),
        ],
        out_specs=pl.BlockSpec((_CR * _COLS, _D), lambda j: (j, 0)),
        out_shape=jax.ShapeDtypeStruct((_N, _D), jnp.float32),
        compiler_params=pltpu.CompilerParams(
            dimension_semantics=("arbitrary",),
        ),
    )(sel, table40)


def kernel(block_type_grid, mask_0, mask_1, mask_2, mask_3,
           values_0, values_1, values_2, values_3,
           table_0, table_1, table_2, table_3):
    table40 = jnp.concatenate(
        [table_0, table_1, table_2, table_3,
         jnp.zeros((_TBL_ROWS - sum(_NUM_EMB), _D), jnp.float32)])

    masks = [m.reshape(_ROWS, _COLS)
             for m in (mask_0, mask_1, mask_2, mask_3)]
    rank, vals_pad = _ranks(masks,
                            (values_0, values_1, values_2, values_3))
    sel = _sc_gather(vals_pad, rank)
    out = _expand(sel, table40)
    return out.reshape(_B, _W, _H, _L, _D)


# speedup vs baseline: 1.0034x; 1.0034x over previous
import dataclasses

import jax
import jax.numpy as jnp
from jax import lax
from jax.experimental import pallas as pl
from jax.experimental.pallas import tpu as pltpu
from jax.experimental.pallas import tpu_sc as plsc

_B, _W, _H, _L = 4, 32, 32, 32
_N = _B * _W * _H * _L
_D = 256
_NUM_EMB = (16, 8, 4, 6)
_OFFS = (0, 16, 24, 28)
_TBL_ROWS = 40
_SENT_ROW = 34

_COUNT = 13107
_VPAD = 13312
_SENT_SLOT = 13200

_ROWS, _COLS = 512, 256

_NC, _NS, _LANES = 2, 16, 16
_NW = _NC * _NS
_TPH = _NW // 4
_TROWS = _ROWS // _TPH


def _rank_body(m0_ref, m1_ref, m2_ref, m3_ref,
               v0_ref, v1_ref, v2_ref, v3_ref, rank_ref, vp_ref):
    iota_r = lax.broadcasted_iota(jnp.int32, (_COLS, _COLS), 0)
    iota_c = lax.broadcasted_iota(jnp.int32, (_COLS, _COLS), 1)
    upper = (iota_r <= iota_c).astype(jnp.bfloat16)
    iota_r2 = lax.broadcasted_iota(jnp.int32, (_ROWS, _ROWS), 0)
    iota_c2 = lax.broadcasted_iota(jnp.int32, (_ROWS, _ROWS), 1)
    strict_lower = (iota_c2 < iota_r2).astype(jnp.bfloat16)

    masks = [m0_ref[...], m1_ref[...], m2_ref[...], m3_ref[...]]
    m_all = jnp.concatenate([m.astype(jnp.bfloat16) for m in masks], axis=0)
    row_incl_all = lax.dot(m_all, upper,
                           preferred_element_type=jnp.float32)
    tot_cols = jnp.concatenate(
        [row_incl_all[i * _ROWS:(i + 1) * _ROWS, _COLS - 1:_COLS]
         for i in range(4)], axis=1)
    hi = jnp.floor(tot_cols * (1.0 / 256.0))
    lo = tot_cols - hi * 256.0
    hilo = jnp.concatenate([hi, lo], axis=1).astype(jnp.bfloat16)
    scan8 = lax.dot(strict_lower, hilo,
                    preferred_element_type=jnp.float32)
    col_excl_all = scan8[:, :4] * 256.0 + scan8[:, 4:]
    for i, (m, v_ref) in enumerate(zip(masks, (v0_ref, v1_ref, v2_ref,
                                               v3_ref))):
        row_incl = row_incl_all[i * _ROWS:(i + 1) * _ROWS]
        excl = row_incl - m.astype(jnp.float32) + col_excl_all[:, i:i + 1]
        rank = excl.astype(jnp.int32)
        rank_ref[i] = jnp.where(m, rank, _SENT_SLOT)
        vp_ref[i] = jnp.concatenate(
            [v_ref[...],
             jnp.full((_VPAD - _COUNT,), _SENT_ROW - _OFFS[i], jnp.int32)])


def _ranks(masks, values):
    return pl.pallas_call(
        _rank_body,
        out_shape=(jax.ShapeDtypeStruct((4, _ROWS, _COLS), jnp.int32),
                   jax.ShapeDtypeStruct((4, _VPAD), jnp.int32)),
    )(*masks, *values)


def _sc_compiler_params():
    cp = pltpu.CompilerParams()
    if "needs_layout_passes" in pltpu.CompilerParams.__dataclass_fields__:
        cp = dataclasses.replace(cp, needs_layout_passes=False)
    return cp


def _gather_body(vals_hbm, idx_hbm, out_hbm, vals_v, idx_v, out_v, sem, sem2):
    wid = lax.axis_index("s") * _NC + lax.axis_index("c")
    head = wid // _TPH
    row0 = (wid % _TPH) * _TROWS
    c1 = pltpu.async_copy(vals_hbm.at[head], vals_v, sem)
    c2 = pltpu.async_copy(idx_hbm.at[head, pl.ds(row0, _TROWS)], idx_v, sem2)
    c1.wait()
    c2.wait()

    @plsc.parallel_loop(0, _TROWS, unroll=4)
    def _(r):
        for c in range(_COLS // _LANES):
            sl = pl.ds(c * _LANES, _LANES)
            out_v[r, sl] = plsc.load_gather(vals_v, [idx_v[r, sl]])

    pltpu.async_copy(out_v, out_hbm.at[head, pl.ds(row0, _TROWS)], sem).wait()


def _sc_gather(vals_pad, rank):
    mesh = plsc.VectorSubcoreMesh(core_axis_name="c", subcore_axis_name="s")
    k = pl.kernel(
        _gather_body,
        out_type=jax.ShapeDtypeStruct((4, _ROWS, _COLS), jnp.int32),
        mesh=mesh,
        scratch_types=[
            pltpu.VMEM((_VPAD,), jnp.int32),
            pltpu.VMEM((_TROWS, _COLS), jnp.int32),
            pltpu.VMEM((_TROWS, _COLS), jnp.int32),
            pltpu.SemaphoreType.DMA,
            pltpu.SemaphoreType.DMA,
        ],
        compiler_params=_sc_compiler_params(),
    )
    return k(vals_pad, rank)


_CR = 32


def _expand_body(sel_ref, tbl_ref, out_ref):
    iota40 = lax.broadcasted_iota(jnp.int32, (_TBL_ROWS, _COLS), 0)
    tbl = tbl_ref[...]
    for r in range(_CR):
        st = None
        for i in range(4):
            sel = (sel_ref[i, r, :] + _OFFS[i]).reshape(1, _COLS)
            eq = jnp.broadcast_to(sel, (_TBL_ROWS, _COLS)) == iota40
            st = eq.astype(jnp.int32) if st is None else st + eq
        tile = lax.dot_general(st.astype(jnp.float32), tbl,
                               (((0,), (0,)), ((), ())),
                               preferred_element_type=jnp.float32)
        out_ref[pl.ds(r * _COLS, _COLS), :] = tile


def _expand(sel, table40):
    return pl.pallas_call(
        _expand_body,
        grid=(_ROWS // _CR,),
        in_specs=[
            pl.BlockSpec((4, _CR, _COLS), lambda j: (0, j, 0)),
            pl.BlockSpec((_TBL_ROWS, _D), lambda j: (0, 0)),
        ],
        out_specs=pl.BlockSpec((_CR * _COLS, _D), lambda j: (j, 0)),
        out_shape=jax.ShapeDtypeStruct((_N, _D), jnp.float32),
        compiler_params=pltpu.CompilerParams(
            dimension_semantics=("arbitrary",),
        ),
    )(sel, table40)


def kernel(block_type_grid, mask_0, mask_1, mask_2, mask_3,
           values_0, values_1, values_2, values_3,
           table_0, table_1, table_2, table_3):
    table40 = jnp.concatenate(
        [table_0, table_1, table_2, table_3,
         jnp.zeros((_TBL_ROWS - sum(_NUM_EMB), _D), jnp.float32)])

    masks = [m.reshape(_ROWS, _COLS)
             for m in (mask_0, mask_1, mask_2, mask_3)]
    rank, vals_pad = _ranks(masks,
                            (values_0, values_1, values_2, values_3))
    sel = _sc_gather(vals_pad, rank)
    out = _expand(sel, table40)
    return out.reshape(_B, _W, _H, _L, _D)
